# baseline (speedup 1.0000x reference)
import functools

import jax
import jax.numpy as jnp
from jax import lax
from jax.experimental import pallas as pl
from jax.experimental.pallas import tpu as pltpu
from jax.experimental.pallas import tpu_sc as plsc

N = 10000
E = 320000
D = 128
DH = D // 2
NC = 2
NS = 16
LANES = 16
EPW = E // NS
K = 80
NCH = EPW // K
ROW_STEP = 624
ROW_CNT = 640

_MESH = plsc.VectorSubcoreMesh(
    core_axis_name="c", subcore_axis_name="s", num_cores=NC, num_subcores=NS
)


@functools.partial(
    pl.kernel,
    out_type=jax.ShapeDtypeStruct((N, D), jnp.float32),
    mesh=_MESH,
    compiler_params=pltpu.CompilerParams(
        needs_layout_passes=False, use_tc_tiling_on_sc=False
    ),
    scratch_types=[
        pltpu.VMEM((NCH, K), jnp.int32),
        pltpu.VMEM((NCH, K), jnp.int32),
        pltpu.VMEM((EPW,), jnp.float32),
        pltpu.VMEM((3, K, DH), jnp.float32),
        pltpu.VMEM_SHARED((N, DH), jnp.float32),
        pltpu.SemaphoreType.DMA,
        pltpu.SemaphoreType.DMA,
        pltpu.SemaphoreType.DMA,
        pltpu.SemaphoreType.DMA,
        pltpu.SemaphoreType.DMA,
        pltpu.SemaphoreType.DMA,
    ],
)
def _sc_aggregate(
    emb2, srcw, dstw, attw, out,
    src_v, dst_v, att_v, rows3_v, acc, g0, g1, g2, s0, s1, s2,
):
    cid = lax.axis_index("c")
    sid = lax.axis_index("s")
    gsems = (g0, g1, g2)
    ssems = (s0, s1, s2)

    pltpu.sync_copy(srcw.at[sid], src_v)
    pltpu.sync_copy(dstw.at[sid], dst_v)
    pltpu.sync_copy(attw.at[sid], att_v)

    zero16 = jnp.zeros((LANES,), jnp.float32)
    zbuf = rows3_v.at[0]

    def _memset_row(e, carry):
        for j in range(DH // LANES):
            zbuf.at[e][pl.ds(j * LANES, LANES)] = zero16
        return carry

    lax.fori_loop(0, K, _memset_row, None)
    row0 = sid * ROW_STEP
    for t in range(ROW_CNT // K):
        pltpu.sync_copy(zbuf, acc.at[pl.ds(row0 + t * K, K)])
    plsc.subcore_barrier()

    cidv = jnp.full((LANES,), cid * N, jnp.int32)

    def _issue_gather(c, b):
        rr = src_v.at[c]
        for j in range(K // LANES):
            sl = pl.ds(j * LANES, LANES)
            v = rr[sl]
            rr[sl] = v + cidv
        pltpu.async_copy(emb2.at[src_v.at[c]], rows3_v.at[b], gsems[b])

    def _wait_gather(c, b):
        pltpu.make_async_copy(emb2.at[src_v.at[c]], rows3_v.at[b], gsems[b]).wait()

    def _wait_scatter(c, b):
        pltpu.make_async_copy(rows3_v.at[b], acc.at[dst_v.at[c]], ssems[b]).wait()

    def _do_chunk(c, b, wait_prev, issue_next):
        _wait_gather(c, b)
        rows_b = rows3_v.at[b]
        cvec = jnp.full((LANES,), c * K, jnp.int32)

        @plsc.parallel_loop(0, K, unroll=4)
        def _edge(e, carry=None):
            att_s = plsc.load_gather(att_v, [cvec + e])
            row = rows_b.at[e]
            for j in range(DH // LANES):
                sl = pl.ds(j * LANES, LANES)
                row[sl] = row[sl] * att_s

        pltpu.async_copy(rows_b, acc.at[dst_v.at[c]], ssems[b], add=True)
        bn = (b + 2) % 3
        if wait_prev:
            _wait_scatter(c - 1, bn)
        if issue_next:
            _issue_gather(c + 2, bn)

    _issue_gather(0, 0)
    _issue_gather(1, 1)
    _do_chunk(0, 0, wait_prev=False, issue_next=True)

    def _round(i, carry):
        c0 = 1 + 3 * i
        _do_chunk(c0, 1, wait_prev=True, issue_next=True)
        _do_chunk(c0 + 1, 2, wait_prev=True, issue_next=True)
        _do_chunk(c0 + 2, 0, wait_prev=True, issue_next=True)
        return carry

    lax.fori_loop(0, (NCH - 4) // 3, _round, None)
    _do_chunk(NCH - 3, 1, wait_prev=True, issue_next=True)
    _do_chunk(NCH - 2, 2, wait_prev=True, issue_next=False)
    _do_chunk(NCH - 1, 0, wait_prev=True, issue_next=False)
    _wait_scatter(NCH - 1, 0)
    plsc.subcore_barrier()

    pltpu.sync_copy(
        acc.at[pl.ds(row0, ROW_CNT)],
        out.at[pl.ds(row0, ROW_CNT), pl.ds(cid * DH, DH)],
    )


def _tc_body(emb_ref, nh_ref, wt_ref, b_ref, out_ref):
    h = emb_ref[...] + nh_ref[...]
    y = jnp.dot(h, wt_ref[...], preferred_element_type=jnp.float32) + b_ref[...]
    out_ref[...] = jnp.where(y >= 0, y, 0.01 * y)


_ROWS_BLK = 400


def _tc_dense(emb, nh, wt, b):
    grid = (N // _ROWS_BLK,)
    return pl.pallas_call(
        _tc_body,
        grid=grid,
        in_specs=[
            pl.BlockSpec((_ROWS_BLK, D), lambda i: (i, 0)),
            pl.BlockSpec((_ROWS_BLK, D), lambda i: (i, 0)),
            pl.BlockSpec((D, D), lambda i: (0, 0)),
            pl.BlockSpec((1, D), lambda i: (0, 0)),
        ],
        out_specs=pl.BlockSpec((_ROWS_BLK, D), lambda i: (i, 0)),
        out_shape=jax.ShapeDtypeStruct((N, D), jnp.float32),
    )(emb, nh, wt, b)


def kernel(entity_embed, edge_att, W, b, edge_index):
    src = edge_index[0].reshape(NS, NCH, K)
    dst = edge_index[1].reshape(NS, NCH, K)
    att = edge_att.reshape(NS, EPW)
    emb2 = entity_embed.reshape(N, NC, DH).transpose(1, 0, 2).reshape(N * NC, DH)

    nh = _sc_aggregate(emb2, src, dst, att)
    out = _tc_dense(entity_embed, nh, W.T, b.reshape(1, D))
    return out


# speedup vs baseline: 1.0769x; 1.0769x over previous
import functools

import jax
import jax.numpy as jnp
from jax import lax
from jax.experimental import pallas as pl
from jax.experimental.pallas import tpu as pltpu
from jax.experimental.pallas import tpu_sc as plsc

N = 10000
E = 320000
D = 128
DH = D // 2
NC = 2
NS = 16
LANES = 16
EPW = E // NS
K = 80
NCH = EPW // K
ROW_STEP = 624
ROW_CNT = 640

_MESH = plsc.VectorSubcoreMesh(
    core_axis_name="c", subcore_axis_name="s", num_cores=NC, num_subcores=NS
)


@functools.partial(
    pl.kernel,
    out_type=jax.ShapeDtypeStruct((N, D), jnp.float32),
    mesh=_MESH,
    compiler_params=pltpu.CompilerParams(
        needs_layout_passes=False, use_tc_tiling_on_sc=False
    ),
    scratch_types=[
        pltpu.VMEM((NCH, K), jnp.int32),
        pltpu.VMEM((NCH, K), jnp.int32),
        pltpu.VMEM((EPW,), jnp.float32),
        pltpu.VMEM((3, K, DH), jnp.float32),
        pltpu.VMEM_SHARED((N, DH), jnp.float32),
        pltpu.SemaphoreType.DMA,
        pltpu.SemaphoreType.DMA,
        pltpu.SemaphoreType.DMA,
        pltpu.SemaphoreType.DMA,
        pltpu.SemaphoreType.DMA,
        pltpu.SemaphoreType.DMA,
    ],
)
def _sc_aggregate(
    emb2, srcw, dstw, attw, out,
    src_v, dst_v, att_v, rows3_v, acc, g0, g1, g2, s0, s1, s2,
):
    cid = lax.axis_index("c")
    sid = lax.axis_index("s")
    gsems = (g0, g1, g2)
    ssems = (s0, s1, s2)

    pltpu.sync_copy(srcw.at[sid], src_v)
    pltpu.sync_copy(dstw.at[sid], dst_v)
    pltpu.sync_copy(attw.at[sid], att_v)

    zero16 = jnp.zeros((LANES,), jnp.float32)
    zbuf = rows3_v.at[0]

    def _memset_row(e, carry):
        for j in range(DH // LANES):
            zbuf.at[e][pl.ds(j * LANES, LANES)] = zero16
        return carry

    lax.fori_loop(0, K, _memset_row, None)
    row0 = sid * ROW_STEP
    for t in range(ROW_CNT // K):
        pltpu.sync_copy(zbuf, acc.at[pl.ds(row0 + t * K, K)])
    plsc.subcore_barrier()

    cidv = jnp.full((LANES,), cid, jnp.int32)

    def _issue_gather(c, b):
        rr = src_v.at[c]
        for j in range(K // LANES):
            sl = pl.ds(j * LANES, LANES)
            v = rr[sl]
            rr[sl] = v + v + cidv
        pltpu.async_copy(emb2.at[src_v.at[c]], rows3_v.at[b], gsems[b])

    def _wait_gather(c, b):
        pltpu.make_async_copy(emb2.at[src_v.at[c]], rows3_v.at[b], gsems[b]).wait()

    def _wait_scatter(c, b):
        pltpu.make_async_copy(rows3_v.at[b], acc.at[dst_v.at[c]], ssems[b]).wait()

    def _do_chunk(c, b, wait_prev, issue_next):
        _wait_gather(c, b)
        rows_b = rows3_v.at[b]
        cvec = jnp.full((LANES,), c * K, jnp.int32)

        @plsc.parallel_loop(0, K, unroll=4)
        def _edge(e, carry=None):
            att_s = plsc.load_gather(att_v, [cvec + e])
            row = rows_b.at[e]
            for j in range(DH // LANES):
                sl = pl.ds(j * LANES, LANES)
                row[sl] = row[sl] * att_s

        pltpu.async_copy(rows_b, acc.at[dst_v.at[c]], ssems[b], add=True)
        bn = (b + 2) % 3
        if wait_prev:
            _wait_scatter(c - 1, bn)
        if issue_next:
            _issue_gather(c + 2, bn)

    _issue_gather(0, 0)
    _issue_gather(1, 1)
    _do_chunk(0, 0, wait_prev=False, issue_next=True)

    def _round(i, carry):
        c0 = 1 + 3 * i
        _do_chunk(c0, 1, wait_prev=True, issue_next=True)
        _do_chunk(c0 + 1, 2, wait_prev=True, issue_next=True)
        _do_chunk(c0 + 2, 0, wait_prev=True, issue_next=True)
        return carry

    lax.fori_loop(0, (NCH - 4) // 3, _round, None)
    _do_chunk(NCH - 3, 1, wait_prev=True, issue_next=True)
    _do_chunk(NCH - 2, 2, wait_prev=True, issue_next=False)
    _do_chunk(NCH - 1, 0, wait_prev=True, issue_next=False)
    _wait_scatter(NCH - 1, 0)
    plsc.subcore_barrier()

    pltpu.sync_copy(
        acc.at[pl.ds(row0, ROW_CNT)],
        out.at[pl.ds(row0, ROW_CNT), pl.ds(cid * DH, DH)],
    )


def _tc_body(emb_ref, nh_ref, wt_ref, b_ref, out_ref):
    h = emb_ref[...] + nh_ref[...]
    y = jnp.dot(h, wt_ref[...], preferred_element_type=jnp.float32) + b_ref[...]
    out_ref[...] = jnp.where(y >= 0, y, 0.01 * y)


_ROWS_BLK = 400


def _tc_dense(emb, nh, wt, b):
    grid = (N // _ROWS_BLK,)
    return pl.pallas_call(
        _tc_body,
        grid=grid,
        in_specs=[
            pl.BlockSpec((_ROWS_BLK, D), lambda i: (i, 0)),
            pl.BlockSpec((_ROWS_BLK, D), lambda i: (i, 0)),
            pl.BlockSpec((D, D), lambda i: (0, 0)),
            pl.BlockSpec((1, D), lambda i: (0, 0)),
        ],
        out_specs=pl.BlockSpec((_ROWS_BLK, D), lambda i: (i, 0)),
        out_shape=jax.ShapeDtypeStruct((N, D), jnp.float32),
    )(emb, nh, wt, b)


def kernel(entity_embed, edge_att, W, b, edge_index):
    src = edge_index[0].reshape(NS, NCH, K)
    dst = edge_index[1].reshape(NS, NCH, K)
    att = edge_att.reshape(NS, EPW)
    emb2 = entity_embed.reshape(N, NC, DH).reshape(N * NC, DH)

    nh = _sc_aggregate(emb2, src, dst, att)
    out = _tc_dense(entity_embed, nh, W.T, b.reshape(1, D))
    return out
